# baseline (speedup 1.0000x reference)
import functools

import jax
import jax.numpy as jnp
from jax import lax
from jax.experimental import pallas as pl
from jax.experimental.pallas import tpu as pltpu
from jax.experimental.pallas import tpu_sc as plsc

H = 384
W = 384
B = 2
C = 96
CP = 128
N = B * H * W
TROWS = 96
SCROWS = H - TROWS
N_SC = B * SCROWS * W
PIX_PER_W = SCROWS * W // 16
CHUNK = 64
NCHUNK = PIX_PER_W // CHUNK
LANES = 16

_INV = 2.0 / W


def _warp_body(src_hbm, fx_hbm, fy_hbm, out_hbm,
               fx_sp, fy_sp, idx_v, wgt_v, taps_v, out_v,
               gsem0, gsem1, osem0, osem1):
    gsems = (gsem0, gsem1)
    osems = (osem0, osem1)
    core = lax.axis_index("c")
    sid = lax.axis_index("s")
    worker_base = core * (H * W) + TROWS * W + sid * PIX_PER_W
    out_base = core * (SCROWS * W) + sid * PIX_PER_W
    pltpu.sync_copy(fx_hbm.at[pl.ds(worker_base, PIX_PER_W)], fx_sp)
    pltpu.sync_copy(fy_hbm.at[pl.ds(worker_base, PIX_PER_W)], fy_sp)

    def stage(ci, b):
        base = worker_base + ci * CHUNK

        def idx_body(j, c2):
            o = j * LANES
            sl = pl.ds(o, LANES)
            fsl = pl.ds(ci * CHUNK + o, LANES)
            row = base // W
            col0 = base % W
            bi = row // H
            yi = row % H
            xi = col0 + o + lax.iota(jnp.int32, LANES)
            gx = (xi.astype(jnp.float32) + 0.5) * _INV - 1.0 + fx_sp[fsl] * _INV
            gy = (jnp.float32(yi) + 0.5) * _INV - 1.0 + fy_sp[fsl] * _INV
            ix = ((gx + 1.0) * W - 1.0) * 0.5
            iy = ((gy + 1.0) * H - 1.0) * 0.5
            ix = jnp.minimum(jnp.maximum(ix, 0.0), jnp.float32(W - 1))
            iy = jnp.minimum(jnp.maximum(iy, 0.0), jnp.float32(H - 1))
            ix0 = ix.astype(jnp.int32)
            iy0 = iy.astype(jnp.int32)
            wx1 = ix - ix0.astype(jnp.float32)
            wy1 = iy - iy0.astype(jnp.float32)
            wx0 = 1.0 - wx1
            wy0 = 1.0 - wy1
            ix1 = jnp.minimum(ix0 + 1, W - 1)
            iy1 = jnp.minimum(iy0 + 1, H - 1)
            row0 = bi * (H * W) + iy0 * W
            row1 = bi * (H * W) + iy1 * W
            idx_v[b, 0, sl] = row0 + ix0
            idx_v[b, 1, sl] = row0 + ix1
            idx_v[b, 2, sl] = row1 + ix0
            idx_v[b, 3, sl] = row1 + ix1
            wgt_v[b, 0, sl] = wy0 * wx0
            wgt_v[b, 1, sl] = wy0 * wx1
            wgt_v[b, 2, sl] = wy1 * wx0
            wgt_v[b, 3, sl] = wy1 * wx1
            return c2

        lax.fori_loop(0, CHUNK // LANES, idx_body, 0, unroll=False)
        for k in range(4):
            pltpu.async_copy(src_hbm.at[idx_v.at[b, k]], taps_v.at[b, k],
                             gsems[b])

    def drain_gathers(b):
        for k in range(4):
            pltpu.make_async_copy(src_hbm.at[idx_v.at[b, k]], taps_v.at[b, k],
                                  gsems[b]).wait()

    def combine(ci, b):
        base = out_base + ci * CHUNK

        def px_body(g, c2):
            gsl = pl.ds(g * LANES, LANES)
            w00v = wgt_v[b, 0, gsl]
            w01v = wgt_v[b, 1, gsl]
            w10v = wgt_v[b, 2, gsl]
            w11v = wgt_v[b, 3, gsl]
            for jj in range(LANES):
                p = g * LANES + jj
                b00 = jnp.full((LANES,), w00v[jj], jnp.float32)
                b01 = jnp.full((LANES,), w01v[jj], jnp.float32)
                b10 = jnp.full((LANES,), w10v[jj], jnp.float32)
                b11 = jnp.full((LANES,), w11v[jj], jnp.float32)
                for cc in range(C // LANES):
                    sl = pl.ds(cc * LANES, LANES)
                    out_v[b, p, sl] = (
                        taps_v[b, 0, p, sl] * b00 + taps_v[b, 1, p, sl] * b01
                        + taps_v[b, 2, p, sl] * b10 + taps_v[b, 3, p, sl] * b11)
            return c2

        lax.fori_loop(0, CHUNK // LANES, px_body, 0, unroll=False)
        pltpu.async_copy(out_v.at[b], out_hbm.at[pl.ds(base, CHUNK)],
                         osems[b])

    def drain_store(b):
        pltpu.make_async_copy(out_v.at[b], out_hbm.at[pl.ds(0, CHUNK)],
                              osems[b]).wait()

    stage(0, 0)

    def pair_body(i, carry):
        for b in range(2):
            ci = 2 * i + b
            nb = 1 - b

            @pl.when(ci + 1 < NCHUNK)
            def _():
                stage(ci + 1, nb)

            drain_gathers(b)

            @pl.when(ci >= 2)
            def _():
                drain_store(b)

            combine(ci, b)
        return carry

    lax.fori_loop(0, NCHUNK // 2, pair_body, 0, unroll=False)
    drain_store(0)
    drain_store(1)


@jax.jit
def _warp(src_cl, fx, fy):
    mesh = plsc.VectorSubcoreMesh(core_axis_name="c", subcore_axis_name="s",
                                  num_cores=2, num_subcores=16)
    f = functools.partial(
        pl.kernel,
        out_type=jax.ShapeDtypeStruct((N_SC, C), jnp.float32),
        mesh=mesh,
        scratch_types=[
            pltpu.VMEM((PIX_PER_W,), jnp.float32),
            pltpu.VMEM((PIX_PER_W,), jnp.float32),
            pltpu.VMEM((2, 4, CHUNK), jnp.int32),
            pltpu.VMEM((2, 4, CHUNK), jnp.float32),
            pltpu.VMEM((2, 4, CHUNK, CP), jnp.float32),
            pltpu.VMEM((2, CHUNK, C), jnp.float32),
            pltpu.SemaphoreType.DMA,
            pltpu.SemaphoreType.DMA,
            pltpu.SemaphoreType.DMA,
            pltpu.SemaphoreType.DMA,
        ],
    )(_warp_body)
    return f(src_cl, fx, fy)


def _warp_rows_tc(src, flow_rows, y0):
    T = flow_rows.shape[1]
    xs = (jnp.arange(W, dtype=jnp.float32) + 0.5) * 2.0 / W - 1.0
    ys = (jnp.arange(y0, y0 + T, dtype=jnp.float32) + 0.5) * 2.0 / H - 1.0
    gx = xs[None, None, :] + flow_rows[..., 0] * (2.0 / W)
    gy = ys[None, :, None] + flow_rows[..., 1] * (2.0 / H)
    ix = jnp.clip(((gx + 1.0) * W - 1.0) * 0.5, 0.0, W - 1.0)
    iy = jnp.clip(((gy + 1.0) * H - 1.0) * 0.5, 0.0, H - 1.0)
    ix0 = jnp.floor(ix)
    iy0 = jnp.floor(iy)
    wx1 = ix - ix0
    wy1 = iy - iy0
    ix0i = jnp.clip(ix0.astype(jnp.int32), 0, W - 1)
    ix1i = jnp.clip(ix0i + 1, 0, W - 1)
    iy0i = jnp.clip(iy0.astype(jnp.int32), 0, H - 1)
    iy1i = jnp.clip(iy0i + 1, 0, H - 1)
    sf = src.reshape(B, C, H * W)
    def gat(yi, xi):
        idx = (yi * W + xi).reshape(B, 1, T * W)
        return jnp.take_along_axis(sf, idx, axis=2).reshape(B, C, T, W)
    w00 = ((1.0 - wy1) * (1.0 - wx1))[:, None]
    w01 = ((1.0 - wy1) * wx1)[:, None]
    w10 = (wy1 * (1.0 - wx1))[:, None]
    w11 = (wy1 * wx1)[:, None]
    return (gat(iy0i, ix0i) * w00 + gat(iy0i, ix1i) * w01
            + gat(iy1i, ix0i) * w10 + gat(iy1i, ix1i) * w11)


def kernel(src, flow):
    src_p = jnp.pad(src, ((0, 0), (0, CP - C), (0, 0), (0, 0)))
    src_cl = src_p.transpose(0, 2, 3, 1).reshape(N, CP)
    fx = flow[..., 0].reshape(N)
    fy = flow[..., 1].reshape(N)
    out_cl = _warp(src_cl, fx, fy)
    out_sc = out_cl.reshape(B, SCROWS, W, C).transpose(0, 3, 1, 2)
    out_tc = _warp_rows_tc(src, flow[:, :TROWS], 0)
    return jnp.concatenate([out_tc, out_sc], axis=2)


# speedup vs baseline: 1.0699x; 1.0699x over previous
import functools

import jax
import jax.numpy as jnp
from jax import lax
from jax.experimental import pallas as pl
from jax.experimental.pallas import tpu as pltpu
from jax.experimental.pallas import tpu_sc as plsc

H = 384
W = 384
B = 2
C = 96
CP = 128
N = B * H * W
NW = 32
PIX_PER_W = N // NW
CHUNK = 64
NCHUNK = PIX_PER_W // CHUNK
LANES = 16

_INV = 2.0 / W


def _warp_body(src_hbm, fx_hbm, fy_hbm, out_hbm,
               fx_sp, fy_sp, idx_v, wgt_v, taps_v, out_v,
               gsem0, gsem1, osem0, osem1):
    gsems = (gsem0, gsem1)
    osems = (osem0, osem1)
    wid = lax.axis_index("s") * 2 + lax.axis_index("c")
    worker_base = wid * PIX_PER_W
    pltpu.sync_copy(fx_hbm.at[pl.ds(worker_base, PIX_PER_W)], fx_sp)
    pltpu.sync_copy(fy_hbm.at[pl.ds(worker_base, PIX_PER_W)], fy_sp)

    def stage(ci, b):
        base = worker_base + ci * CHUNK

        def idx_body(j, c2):
            o = j * LANES
            sl = pl.ds(o, LANES)
            fsl = pl.ds(ci * CHUNK + o, LANES)
            row = base // W
            col0 = base % W
            bi = row // H
            yi = row % H
            xi = col0 + o + lax.iota(jnp.int32, LANES)
            gx = (xi.astype(jnp.float32) + 0.5) * _INV - 1.0 + fx_sp[fsl] * _INV
            gy = (jnp.float32(yi) + 0.5) * _INV - 1.0 + fy_sp[fsl] * _INV
            ix = ((gx + 1.0) * W - 1.0) * 0.5
            iy = ((gy + 1.0) * H - 1.0) * 0.5
            ix = jnp.minimum(jnp.maximum(ix, 0.0), jnp.float32(W - 1))
            iy = jnp.minimum(jnp.maximum(iy, 0.0), jnp.float32(H - 1))
            ix0 = ix.astype(jnp.int32)
            iy0 = iy.astype(jnp.int32)
            wx1 = ix - ix0.astype(jnp.float32)
            wy1 = iy - iy0.astype(jnp.float32)
            wx0 = 1.0 - wx1
            wy0 = 1.0 - wy1
            ix1 = jnp.minimum(ix0 + 1, W - 1)
            iy1 = jnp.minimum(iy0 + 1, H - 1)
            row0 = bi * (H * W) + iy0 * W
            row1 = bi * (H * W) + iy1 * W
            idx_v[b, 0, sl] = row0 + ix0
            idx_v[b, 1, sl] = row0 + ix1
            idx_v[b, 2, sl] = row1 + ix0
            idx_v[b, 3, sl] = row1 + ix1
            wgt_v[b, 0, sl] = wy0 * wx0
            wgt_v[b, 1, sl] = wy0 * wx1
            wgt_v[b, 2, sl] = wy1 * wx0
            wgt_v[b, 3, sl] = wy1 * wx1
            return c2

        lax.fori_loop(0, CHUNK // LANES, idx_body, 0, unroll=True)
        for k in range(4):
            pltpu.async_copy(src_hbm.at[idx_v.at[b, k]], taps_v.at[b, k],
                             gsems[b])

    def drain_gathers(b):
        for k in range(4):
            pltpu.make_async_copy(src_hbm.at[idx_v.at[b, k]], taps_v.at[b, k],
                                  gsems[b]).wait()

    def combine(ci, b):
        base = worker_base + ci * CHUNK

        def px_body(g, c2):
            gsl = pl.ds(g * LANES, LANES)
            w00v = wgt_v[b, 0, gsl]
            w01v = wgt_v[b, 1, gsl]
            w10v = wgt_v[b, 2, gsl]
            w11v = wgt_v[b, 3, gsl]
            for jj in range(LANES):
                p = g * LANES + jj
                b00 = jnp.full((LANES,), w00v[jj], jnp.float32)
                b01 = jnp.full((LANES,), w01v[jj], jnp.float32)
                b10 = jnp.full((LANES,), w10v[jj], jnp.float32)
                b11 = jnp.full((LANES,), w11v[jj], jnp.float32)
                for cc in range(C // LANES):
                    sl = pl.ds(cc * LANES, LANES)
                    out_v[b, p, sl] = (
                        taps_v[b, 0, p, sl] * b00 + taps_v[b, 1, p, sl] * b01
                        + taps_v[b, 2, p, sl] * b10 + taps_v[b, 3, p, sl] * b11)
            return c2

        lax.fori_loop(0, CHUNK // LANES, px_body, 0, unroll=True)
        pltpu.async_copy(out_v.at[b], out_hbm.at[pl.ds(base, CHUNK)],
                         osems[b])

    def drain_store(b):
        pltpu.make_async_copy(out_v.at[b], out_hbm.at[pl.ds(0, CHUNK)],
                              osems[b]).wait()

    stage(0, 0)

    def pair_body(i, carry):
        for b in range(2):
            ci = 2 * i + b
            nb = 1 - b

            @pl.when(ci + 1 < NCHUNK)
            def _():
                stage(ci + 1, nb)

            drain_gathers(b)

            @pl.when(ci >= 2)
            def _():
                drain_store(b)

            combine(ci, b)
        return carry

    lax.fori_loop(0, NCHUNK // 2, pair_body, 0, unroll=False)
    drain_store(0)
    drain_store(1)


@jax.jit
def _warp(src_cl, fx, fy):
    mesh = plsc.VectorSubcoreMesh(core_axis_name="c", subcore_axis_name="s",
                                  num_cores=2, num_subcores=16)
    f = functools.partial(
        pl.kernel,
        out_type=jax.ShapeDtypeStruct((N, C), jnp.float32),
        mesh=mesh,
        scratch_types=[
            pltpu.VMEM((PIX_PER_W,), jnp.float32),
            pltpu.VMEM((PIX_PER_W,), jnp.float32),
            pltpu.VMEM((2, 4, CHUNK), jnp.int32),
            pltpu.VMEM((2, 4, CHUNK), jnp.float32),
            pltpu.VMEM((2, 4, CHUNK, CP), jnp.float32),
            pltpu.VMEM((2, CHUNK, C), jnp.float32),
            pltpu.SemaphoreType.DMA,
            pltpu.SemaphoreType.DMA,
            pltpu.SemaphoreType.DMA,
            pltpu.SemaphoreType.DMA,
        ],
    )(_warp_body)
    return f(src_cl, fx, fy)


def kernel(src, flow):
    src_p = jnp.pad(src, ((0, 0), (0, CP - C), (0, 0), (0, 0)))
    src_cl = src_p.transpose(0, 2, 3, 1).reshape(N, CP)
    fx = flow[..., 0].reshape(N)
    fy = flow[..., 1].reshape(N)
    out_cl = _warp(src_cl, fx, fy)
    return out_cl.reshape(B, H, W, C).transpose(0, 3, 1, 2)


# speedup vs baseline: 1.7128x; 1.6009x over previous
import functools

import jax
import jax.numpy as jnp
from jax import lax
from jax.experimental import pallas as pl
from jax.experimental.pallas import tpu as pltpu
from jax.experimental.pallas import tpu_sc as plsc

H = 384
W = 384
B = 2
C = 96
CP = 128
N = B * H * W
NW = 32
PIX_PER_W = N // NW
CHUNK = 64
NCHUNK = PIX_PER_W // CHUNK
LANES = 16

_INV = 2.0 / W


def _warp_body(src_hbm, fx_hbm, fy_hbm, out_hbm,
               fx_sp, fy_sp, idx_v, wgt_v, taps_v, out_v,
               gsem0, gsem1, osem0, osem1):
    gsems = (gsem0, gsem1)
    osems = (osem0, osem1)
    wid = lax.axis_index("s") * 2 + lax.axis_index("c")
    worker_base = wid * PIX_PER_W
    pltpu.sync_copy(fx_hbm.at[pl.ds(worker_base, PIX_PER_W)], fx_sp)
    pltpu.sync_copy(fy_hbm.at[pl.ds(worker_base, PIX_PER_W)], fy_sp)

    def stage(ci, b):
        base = worker_base + ci * CHUNK

        def idx_body(j, c2):
            o = j * LANES
            sl = pl.ds(o, LANES)
            fsl = pl.ds(ci * CHUNK + o, LANES)
            row = base // W
            col0 = base % W
            bi = row // H
            yi = row % H
            xi = col0 + o + lax.iota(jnp.int32, LANES)
            gx = (xi.astype(jnp.float32) + 0.5) * _INV - 1.0 + fx_sp[fsl] * _INV
            gy = (jnp.float32(yi) + 0.5) * _INV - 1.0 + fy_sp[fsl] * _INV
            ix = ((gx + 1.0) * W - 1.0) * 0.5
            iy = ((gy + 1.0) * H - 1.0) * 0.5
            ix = jnp.minimum(jnp.maximum(ix, 0.0), jnp.float32(W - 1))
            iy = jnp.minimum(jnp.maximum(iy, 0.0), jnp.float32(H - 1))
            ix0 = ix.astype(jnp.int32)
            iy0 = iy.astype(jnp.int32)
            wx1 = ix - ix0.astype(jnp.float32)
            wy1 = iy - iy0.astype(jnp.float32)
            wx0 = 1.0 - wx1
            wy0 = 1.0 - wy1
            ix1 = jnp.minimum(ix0 + 1, W - 1)
            iy1 = jnp.minimum(iy0 + 1, H - 1)
            row0 = bi * (H * W) + iy0 * W
            row1 = bi * (H * W) + iy1 * W
            idx_v[b, 0, sl] = row0 + ix0
            idx_v[b, 1, sl] = row0 + ix1
            idx_v[b, 2, sl] = row1 + ix0
            idx_v[b, 3, sl] = row1 + ix1
            wgt_v[b, 0, sl] = wy0 * wx0
            wgt_v[b, 1, sl] = wy0 * wx1
            wgt_v[b, 2, sl] = wy1 * wx0
            wgt_v[b, 3, sl] = wy1 * wx1
            return c2

        lax.fori_loop(0, CHUNK // LANES, idx_body, 0, unroll=False)
        for k in range(4):
            pltpu.async_copy(src_hbm.at[idx_v.at[b, k]], taps_v.at[b, k],
                             gsems[b])

    def drain_gathers(b):
        for k in range(4):
            pltpu.make_async_copy(src_hbm.at[idx_v.at[b, k]], taps_v.at[b, k],
                                  gsems[b]).wait()

    def combine(ci, b):
        base = worker_base + ci * CHUNK

        def px_body(g, c2):
            gsl = pl.ds(g * LANES, LANES)
            w00v = wgt_v[b, 0, gsl]
            w01v = wgt_v[b, 1, gsl]
            w10v = wgt_v[b, 2, gsl]
            w11v = wgt_v[b, 3, gsl]
            for jj in range(LANES):
                p = g * LANES + jj
                b00 = jnp.full((LANES,), w00v[jj], jnp.float32)
                b01 = jnp.full((LANES,), w01v[jj], jnp.float32)
                b10 = jnp.full((LANES,), w10v[jj], jnp.float32)
                b11 = jnp.full((LANES,), w11v[jj], jnp.float32)
                for cc in range(C // LANES):
                    sl = pl.ds(cc * LANES, LANES)
                    out_v[b, p, sl] = (
                        taps_v[b, 0, p, sl] * b00 + taps_v[b, 1, p, sl] * b01
                        + taps_v[b, 2, p, sl] * b10 + taps_v[b, 3, p, sl] * b11)
            return c2

        lax.fori_loop(0, CHUNK // LANES, px_body, 0, unroll=False)
        pltpu.async_copy(out_v.at[b], out_hbm.at[pl.ds(base, CHUNK)],
                         osems[b])

    def drain_store(b):
        pltpu.make_async_copy(out_v.at[b], out_hbm.at[pl.ds(0, CHUNK)],
                              osems[b]).wait()

    stage(0, 0)

    def pair_body(i, carry):
        for b in range(2):
            ci = 2 * i + b
            nb = 1 - b

            @pl.when(ci + 1 < NCHUNK)
            def _():
                stage(ci + 1, nb)

            drain_gathers(b)

            @pl.when(ci >= 2)
            def _():
                drain_store(b)

            combine(ci, b)
        return carry

    lax.fori_loop(0, NCHUNK // 2, pair_body, 0, unroll=False)
    drain_store(0)
    drain_store(1)


@jax.jit
def _warp(src_cl, fx, fy):
    mesh = plsc.VectorSubcoreMesh(core_axis_name="c", subcore_axis_name="s",
                                  num_cores=2, num_subcores=16)
    f = functools.partial(
        pl.kernel,
        out_type=jax.ShapeDtypeStruct((N, C), jnp.float32),
        mesh=mesh,
        scratch_types=[
            pltpu.VMEM((PIX_PER_W,), jnp.float32),
            pltpu.VMEM((PIX_PER_W,), jnp.float32),
            pltpu.VMEM((2, 4, CHUNK), jnp.int32),
            pltpu.VMEM((2, 4, CHUNK), jnp.float32),
            pltpu.VMEM((2, 4, CHUNK, CP), jnp.float32),
            pltpu.VMEM((2, CHUNK, C), jnp.float32),
            pltpu.SemaphoreType.DMA,
            pltpu.SemaphoreType.DMA,
            pltpu.SemaphoreType.DMA,
            pltpu.SemaphoreType.DMA,
        ],
    )(_warp_body)
    return f(src_cl, fx, fy)


def kernel(src, flow):
    src_p = jnp.pad(src, ((0, 0), (0, CP - C), (0, 0), (0, 0)))
    src_cl = src_p.transpose(0, 2, 3, 1).reshape(N, CP)
    fx = flow[..., 0].reshape(N)
    fy = flow[..., 1].reshape(N)
    out_cl = _warp(src_cl, fx, fy)
    return out_cl.reshape(B, H, W, C).transpose(0, 3, 1, 2)
